# baseline (speedup 1.0000x reference)
import functools

import jax
import jax.numpy as jnp
from jax import lax
from jax.experimental import pallas as pl
from jax.experimental.pallas import tpu as pltpu
from jax.experimental.pallas import tpu_sc as plsc

BATCH = 16384
EMBED_DIM = 128

NUM_CORES = 2
NUM_SUBCORES = 16
NUM_WORKERS = NUM_CORES * NUM_SUBCORES
B_PER_W = BATCH // NUM_WORKERS
IDX_CHUNK = 128
N_CHUNKS = B_PER_W // IDX_CHUNK

_MESH = plsc.VectorSubcoreMesh(
    core_axis_name="c", subcore_axis_name="s",
    num_cores=NUM_CORES, num_subcores=NUM_SUBCORES,
)


def _gather_body(idx_hbm, table_hbm, out_hbm, idx_v, rows_v, sem, wsem):
    wid = lax.axis_index("s") * NUM_CORES + lax.axis_index("c")
    base = wid * B_PER_W
    pltpu.sync_copy(idx_hbm.at[wid], idx_v)
    gathers = []
    for j in range(N_CHUNKS):
        gathers.append(pltpu.async_copy(
            table_hbm.at[idx_v.at[j]],
            rows_v.at[pl.ds(j * IDX_CHUNK, IDX_CHUNK)],
            sem,
        ))
    writes = []
    for j in range(N_CHUNKS):
        gathers[j].wait()
        writes.append(pltpu.async_copy(
            rows_v.at[pl.ds(j * IDX_CHUNK, IDX_CHUNK)],
            out_hbm.at[pl.ds(base + j * IDX_CHUNK, IDX_CHUNK)],
            wsem,
        ))
    for w in writes:
        w.wait()


@functools.partial(jax.jit, static_argnames=())
def _sc_gather(idx, table):
    call = pl.kernel(
        _gather_body,
        out_type=jax.ShapeDtypeStruct((BATCH, EMBED_DIM), jnp.float32),
        mesh=_MESH,
        scratch_types=[
            pltpu.VMEM((N_CHUNKS, IDX_CHUNK), jnp.int32),
            pltpu.VMEM((B_PER_W, EMBED_DIM), jnp.float32),
            pltpu.SemaphoreType.DMA,
            pltpu.SemaphoreType.DMA,
        ],
    )
    return call(idx, table)


def kernel(x, indices, embedding_weight):
    del x
    idx = indices.astype(jnp.int32).reshape(NUM_WORKERS, N_CHUNKS, IDX_CHUNK)
    out = _sc_gather(idx, embedding_weight)
    return out[:, None, :]


# speedup vs baseline: 1.0007x; 1.0007x over previous
import functools

import jax
import jax.numpy as jnp
from jax import lax
from jax.experimental import pallas as pl
from jax.experimental.pallas import tpu as pltpu
from jax.experimental.pallas import tpu_sc as plsc

BATCH = 16384
EMBED_DIM = 128

NUM_CORES = 2
NUM_SUBCORES = 16
NUM_WORKERS = NUM_CORES * NUM_SUBCORES
B_PER_W = BATCH // NUM_WORKERS
IDX_CHUNK = 128
N_CHUNKS = B_PER_W // IDX_CHUNK

_MESH = plsc.VectorSubcoreMesh(
    core_axis_name="c", subcore_axis_name="s",
    num_cores=NUM_CORES, num_subcores=NUM_SUBCORES,
)


def _gather_body(idx_hbm, table_hbm, out_hbm, idx_v, rows_v, sem, wsem):
    wid = lax.axis_index("s") * NUM_CORES + lax.axis_index("c")
    base = wid * B_PER_W
    pltpu.sync_copy(idx_hbm.at[wid], idx_v)

    def fire(j, c):
        pltpu.async_copy(
            table_hbm.at[idx_v.at[j]],
            rows_v.at[pl.ds(j * IDX_CHUNK, IDX_CHUNK)],
            sem,
        )
        return c

    def drain_and_write(j, c):
        chunk = rows_v.at[pl.ds(j * IDX_CHUNK, IDX_CHUNK)]
        pltpu.make_async_copy(table_hbm.at[idx_v.at[j]], chunk, sem).wait()
        pltpu.async_copy(
            chunk, out_hbm.at[pl.ds(base + j * IDX_CHUNK, IDX_CHUNK)], wsem)
        return c

    lax.fori_loop(0, N_CHUNKS, fire, 0)
    lax.fori_loop(0, N_CHUNKS, drain_and_write, 0)
    pltpu.make_async_copy(out_hbm.at[pl.ds(base, B_PER_W)], rows_v, wsem).wait()


@functools.partial(jax.jit, static_argnames=())
def _sc_gather(idx, table):
    call = pl.kernel(
        _gather_body,
        out_type=jax.ShapeDtypeStruct((BATCH, EMBED_DIM), jnp.float32),
        mesh=_MESH,
        scratch_types=[
            pltpu.VMEM((N_CHUNKS, IDX_CHUNK), jnp.int32),
            pltpu.VMEM((B_PER_W, EMBED_DIM), jnp.float32),
            pltpu.SemaphoreType.DMA,
            pltpu.SemaphoreType.DMA,
        ],
    )
    return call(idx, table)


def kernel(x, indices, embedding_weight):
    del x
    idx = indices.astype(jnp.int32).reshape(NUM_WORKERS, N_CHUNKS, IDX_CHUNK)
    out = _sc_gather(idx, embedding_weight)
    return out[:, None, :]


# speedup vs baseline: 1.0209x; 1.0202x over previous
import functools

import jax
import jax.numpy as jnp
from jax import lax
from jax.experimental import pallas as pl
from jax.experimental.pallas import tpu as pltpu
from jax.experimental.pallas import tpu_sc as plsc

BATCH = 16384
EMBED_DIM = 128

NUM_CORES = 2
NUM_SUBCORES = 16
NUM_WORKERS = NUM_CORES * NUM_SUBCORES
B_PER_W = BATCH // NUM_WORKERS
IDX_CHUNK = 128
N_CHUNKS = B_PER_W // IDX_CHUNK

_MESH = plsc.VectorSubcoreMesh(
    core_axis_name="c", subcore_axis_name="s",
    num_cores=NUM_CORES, num_subcores=NUM_SUBCORES,
)


def _gather_body(idx_hbm, table_hbm, out_hbm, idx_v, rows_v, sem):
    wid = lax.axis_index("s") * NUM_CORES + lax.axis_index("c")
    base = wid * B_PER_W
    pltpu.sync_copy(idx_hbm.at[wid], idx_v)

    def fire(j, c):
        pltpu.async_copy(
            table_hbm.at[idx_v.at[j]],
            rows_v.at[pl.ds(j * IDX_CHUNK, IDX_CHUNK)],
            sem,
        )
        return c

    lax.fori_loop(0, N_CHUNKS, fire, 0)
    pltpu.make_async_copy(out_hbm.at[pl.ds(base, B_PER_W)], rows_v, sem).wait()
    pltpu.sync_copy(rows_v, out_hbm.at[pl.ds(base, B_PER_W)])


@functools.partial(jax.jit, static_argnames=())
def _sc_gather(idx, table):
    call = pl.kernel(
        _gather_body,
        out_type=jax.ShapeDtypeStruct((BATCH, EMBED_DIM), jnp.float32),
        mesh=_MESH,
        scratch_types=[
            pltpu.VMEM((N_CHUNKS, IDX_CHUNK), jnp.int32),
            pltpu.VMEM((B_PER_W, EMBED_DIM), jnp.float32),
            pltpu.SemaphoreType.DMA,
        ],
    )
    return call(idx, table)


def kernel(x, indices, embedding_weight):
    del x
    idx = indices.astype(jnp.int32).reshape(NUM_WORKERS, N_CHUNKS, IDX_CHUNK)
    out = _sc_gather(idx, embedding_weight)
    return out[:, None, :]


# speedup vs baseline: 1.0230x; 1.0021x over previous
import functools

import jax
import jax.numpy as jnp
from jax import lax
from jax.experimental import pallas as pl
from jax.experimental.pallas import tpu as pltpu
from jax.experimental.pallas import tpu_sc as plsc

BATCH = 16384
EMBED_DIM = 128

NUM_CORES = 2
NUM_SUBCORES = 16
NUM_WORKERS = NUM_CORES * NUM_SUBCORES
B_PER_W = BATCH // NUM_WORKERS
IDX_CHUNK = 128
N_CHUNKS = B_PER_W // IDX_CHUNK

_MESH = plsc.VectorSubcoreMesh(
    core_axis_name="c", subcore_axis_name="s",
    num_cores=NUM_CORES, num_subcores=NUM_SUBCORES,
)


def _gather_body(idx_hbm, table_hbm, out_hbm, idx_v, rows_v, sem):
    wid = lax.axis_index("s") * NUM_CORES + lax.axis_index("c")
    base = wid * B_PER_W
    pltpu.sync_copy(idx_hbm.at[wid], idx_v)

    def fire(j, c):
        pltpu.async_copy(
            table_hbm.at[idx_v.at[j]],
            rows_v.at[pl.ds(j * IDX_CHUNK, IDX_CHUNK)],
            sem,
        )
        return c

    lax.fori_loop(0, N_CHUNKS, fire, 0)
    pltpu.make_async_copy(out_hbm.at[pl.ds(base, B_PER_W)], rows_v, sem).wait()
    pltpu.sync_copy(rows_v, out_hbm.at[pl.ds(base, B_PER_W)])


@functools.partial(jax.jit, static_argnames=())
def _sc_gather(idx, table):
    call = pl.kernel(
        _gather_body,
        out_type=jax.ShapeDtypeStruct((BATCH, EMBED_DIM), jnp.float32),
        mesh=_MESH,
        scratch_types=[
            pltpu.VMEM((N_CHUNKS, IDX_CHUNK), jnp.int32),
            pltpu.VMEM((B_PER_W, EMBED_DIM), jnp.float32),
            pltpu.SemaphoreType.DMA,
        ],
        compiler_params=pltpu.CompilerParams(skip_device_barrier=True),
    )
    return call(idx, table)


def kernel(x, indices, embedding_weight):
    del x
    idx = indices.astype(jnp.int32).reshape(NUM_WORKERS, N_CHUNKS, IDX_CHUNK)
    out = _sc_gather(idx, embedding_weight)
    return out[:, None, :]
